# Initial kernel scaffold; baseline (speedup 1.0000x reference)
import jax
import jax.numpy as jnp
from jax.experimental import pallas as pl


def kernel(x, edge_index, W1, b1, W2, b2, W_out, b_out):
    raise NotImplementedError("write your pallas kernel here")


# speedup vs baseline: 17.7271x; 17.7271x over previous
import jax
import jax.numpy as jnp
from jax import lax
from jax.experimental import pallas as pl
from jax.experimental.pallas import tpu as pltpu
from jax.experimental.pallas import tpu_sc as plsc

N = 10000
D = 128
HALF = D // 2
C_OUT = 40
E = 320000

NC = 2
NS = 16
L = 16

EPT = E // NS
CHUNK = 400
NCHUNK = EPT // CHUNK

DEPT = E // (NC * NS)
DCHUNK = 2000
NDCHUNK = DEPT // DCHUNK


def _deg_body(dst_hbm, out_hbm, didx_v, ones_v, deg_sh):
    cid = lax.axis_index("c")
    sid = lax.axis_index("s")
    wid = cid * NS + sid

    def _fill_zero(i, _):
        ones_v[pl.ds(i * L, L)] = jnp.zeros((L,), jnp.float32)
        return 0

    def _fill_one(i, _):
        ones_v[pl.ds(i * L, L)] = jnp.ones((L,), jnp.float32)
        return 0

    lax.fori_loop(0, DCHUNK // L, _fill_zero, 0)

    @pl.when(sid == 0)
    def _zero_shared():
        for c in range(N // DCHUNK):
            pltpu.sync_copy(ones_v, deg_sh.at[pl.ds(c * DCHUNK, DCHUNK)])

    lax.fori_loop(0, DCHUNK // L, _fill_one, 0)
    plsc.subcore_barrier()

    def _chunk(c, _):
        base = wid * DEPT + c * DCHUNK
        pltpu.sync_copy(dst_hbm.at[pl.ds(base, DCHUNK)], didx_v)
        pltpu.sync_copy(ones_v, deg_sh.at[didx_v], add=True)
        return 0

    lax.fori_loop(0, NDCHUNK, _chunk, 0)
    plsc.subcore_barrier()

    @pl.when(sid == 0)
    def _flush():
        pltpu.sync_copy(deg_sh, out_hbm.at[cid])


_deg_kernel = pl.kernel(
    _deg_body,
    out_type=jax.ShapeDtypeStruct((NC, N), jnp.float32),
    mesh=plsc.VectorSubcoreMesh(
        core_axis_name="c", subcore_axis_name="s", num_cores=NC,
        num_subcores=NS),
    scratch_types=[
        pltpu.VMEM((DCHUNK,), jnp.int32),
        pltpu.VMEM((DCHUNK,), jnp.float32),
        pltpu.VMEM_SHARED((N,), jnp.float32),
    ],
    compiler_params=pltpu.CompilerParams(use_tc_tiling_on_sc=False),
)


def _agg_body(g_hbm, src_hbm, dst_hbm, out_hbm, sidx, didx, rows, acc_sh,
              sem):
    cid = lax.axis_index("c")
    sid = lax.axis_index("s")

    def _zrow(i, _):
        rows[i // 4, pl.ds((i % 4) * L, L)] = jnp.zeros((L,), jnp.float32)
        return 0

    lax.fori_loop(0, 16 * (HALF // L), _zrow, 0)

    def _zchunk(j, _):
        chunk = sid + j * NS

        @pl.when(chunk < N // 16)
        def _():
            off = pl.multiple_of(chunk * 16, 16)
            pltpu.sync_copy(rows.at[pl.ds(0, 16)],
                            acc_sh.at[pl.ds(off, 16)])

        return 0

    lax.fori_loop(0, pl.cdiv(N // 16, NS), _zchunk, 0)
    plsc.subcore_barrier()

    def _chunk(k, _):
        base = sid * EPT + k * CHUNK
        pltpu.sync_copy(src_hbm.at[pl.ds(base, CHUNK)], sidx)
        pltpu.sync_copy(dst_hbm.at[pl.ds(base, CHUNK)], didx)
        pltpu.async_copy(g_hbm.at[cid].at[sidx], rows, sem).wait()
        pltpu.sync_copy(rows, acc_sh.at[didx], add=True)
        return 0

    lax.fori_loop(0, NCHUNK, _chunk, 0)
    plsc.subcore_barrier()

    def _fchunk(j, _):
        chunk = sid + j * NS

        @pl.when(chunk < N // 16)
        def _():
            off = pl.multiple_of(chunk * 16, 16)
            pltpu.sync_copy(acc_sh.at[pl.ds(off, 16)],
                            out_hbm.at[cid, pl.ds(off, 16)])

        return 0

    lax.fori_loop(0, pl.cdiv(N // 16, NS), _fchunk, 0)


_agg_kernel = pl.kernel(
    _agg_body,
    out_type=jax.ShapeDtypeStruct((NC, N, HALF), jnp.float32),
    mesh=plsc.VectorSubcoreMesh(
        core_axis_name="c", subcore_axis_name="s", num_cores=NC,
        num_subcores=NS),
    scratch_types=[
        pltpu.VMEM((CHUNK,), jnp.int32),
        pltpu.VMEM((CHUNK,), jnp.int32),
        pltpu.VMEM((CHUNK, HALF), jnp.float32),
        pltpu.VMEM_SHARED((N, HALF), jnp.float32),
        pltpu.SemaphoreType.DMA,
    ],
    compiler_params=pltpu.CompilerParams(use_tc_tiling_on_sc=False),
)

MB = 2000
GRID = N // MB


def _k1_body(x_ref, w_ref, degp_ref, g_ref, dis_ref):
    deg = degp_ref[0] + degp_ref[1] + 1.0
    dis = lax.rsqrt(deg)
    h = jnp.dot(x_ref[...], w_ref[...],
                preferred_element_type=jnp.float32) * dis
    g_ref[0] = h[:, :HALF]
    g_ref[1] = h[:, HALF:]
    dis_ref[...] = dis


_k1 = pl.pallas_call(
    _k1_body,
    grid=(GRID,),
    in_specs=[
        pl.BlockSpec((MB, D), lambda i: (i, 0)),
        pl.BlockSpec((D, D), lambda i: (0, 0)),
        pl.BlockSpec((NC, MB, 1), lambda i: (0, i, 0)),
    ],
    out_specs=[
        pl.BlockSpec((NC, MB, HALF), lambda i: (0, i, 0)),
        pl.BlockSpec((MB, 1), lambda i: (i, 0)),
    ],
    out_shape=[
        jax.ShapeDtypeStruct((NC, N, HALF), jnp.float32),
        jax.ShapeDtypeStruct((N, 1), jnp.float32),
    ],
)


def _k2_body(s_ref, g_ref, dis_ref, b_ref, w_ref, g2_ref):
    sg = jnp.concatenate([s_ref[0] + g_ref[0], s_ref[1] + g_ref[1]],
                         axis=-1)
    t = jnp.maximum(sg * dis_ref[...] + b_ref[...], 0.0)
    h = jnp.dot(t, w_ref[...],
                preferred_element_type=jnp.float32) * dis_ref[...]
    g2_ref[0] = h[:, :HALF]
    g2_ref[1] = h[:, HALF:]


_k2 = pl.pallas_call(
    _k2_body,
    grid=(GRID,),
    in_specs=[
        pl.BlockSpec((NC, MB, HALF), lambda i: (0, i, 0)),
        pl.BlockSpec((NC, MB, HALF), lambda i: (0, i, 0)),
        pl.BlockSpec((MB, 1), lambda i: (i, 0)),
        pl.BlockSpec((1, D), lambda i: (0, 0)),
        pl.BlockSpec((D, D), lambda i: (0, 0)),
    ],
    out_specs=pl.BlockSpec((NC, MB, HALF), lambda i: (0, i, 0)),
    out_shape=jax.ShapeDtypeStruct((NC, N, HALF), jnp.float32),
)


def _k3_body(s_ref, g_ref, dis_ref, b_ref, w_ref, bo_ref, out_ref):
    sg = jnp.concatenate([s_ref[0] + g_ref[0], s_ref[1] + g_ref[1]],
                         axis=-1)
    t = sg * dis_ref[...] + b_ref[...]
    out_ref[...] = jnp.dot(
        t, w_ref[...], preferred_element_type=jnp.float32) + bo_ref[...]


_k3 = pl.pallas_call(
    _k3_body,
    grid=(GRID,),
    in_specs=[
        pl.BlockSpec((NC, MB, HALF), lambda i: (0, i, 0)),
        pl.BlockSpec((NC, MB, HALF), lambda i: (0, i, 0)),
        pl.BlockSpec((MB, 1), lambda i: (i, 0)),
        pl.BlockSpec((1, D), lambda i: (0, 0)),
        pl.BlockSpec((D, D), lambda i: (0, 0)),
        pl.BlockSpec((1, D), lambda i: (0, 0)),
    ],
    out_specs=pl.BlockSpec((MB, D), lambda i: (i, 0)),
    out_shape=jax.ShapeDtypeStruct((N, D), jnp.float32),
)


def kernel(x, edge_index, W1, b1, W2, b2, W_out, b_out):
    src = edge_index[0].astype(jnp.int32)
    dst = edge_index[1].astype(jnp.int32)

    degp = _deg_kernel(dst).reshape(NC, N, 1)
    g1, dis = _k1(x, W1, degp)
    s1 = _agg_kernel(g1, src, dst)
    g2 = _k2(s1, g1, dis, b1.reshape(1, D), W2)
    s2 = _agg_kernel(g2, src, dst)

    W_out_p = jnp.zeros((D, D), jnp.float32).at[:, :C_OUT].set(W_out)
    b_out_p = jnp.zeros((1, D), jnp.float32).at[0, :C_OUT].set(b_out)
    out = _k3(s2, g2, dis, b2.reshape(1, D), W_out_p, b_out_p)
    return out[:, :C_OUT]


# speedup vs baseline: 26.5573x; 1.4981x over previous
import jax
import jax.numpy as jnp
from jax import lax
from jax.experimental import pallas as pl
from jax.experimental.pallas import tpu as pltpu
from jax.experimental.pallas import tpu_sc as plsc

N = 10000
D = 128
HALF = D // 2
C_OUT = 40
E = 320000

NC = 2
NS = 16
L = 16

EPT = E // NS
CHUNK = 400
NCHUNK = EPT // CHUNK

DEPT = E // (NC * NS)
DCHUNK = 2000
NDCHUNK = DEPT // DCHUNK


def _deg_body(dst_hbm, out_hbm, didx_v, ones_v, deg_sh):
    cid = lax.axis_index("c")
    sid = lax.axis_index("s")
    wid = cid * NS + sid

    def _fill_zero(i, _):
        ones_v[pl.ds(i * L, L)] = jnp.zeros((L,), jnp.float32)
        return 0

    def _fill_one(i, _):
        ones_v[pl.ds(i * L, L)] = jnp.ones((L,), jnp.float32)
        return 0

    lax.fori_loop(0, DCHUNK // L, _fill_zero, 0)

    @pl.when(sid == 0)
    def _zero_shared():
        for c in range(N // DCHUNK):
            pltpu.sync_copy(ones_v, deg_sh.at[pl.ds(c * DCHUNK, DCHUNK)])

    lax.fori_loop(0, DCHUNK // L, _fill_one, 0)
    plsc.subcore_barrier()

    def _chunk(c, _):
        base = wid * DEPT + c * DCHUNK
        pltpu.sync_copy(dst_hbm.at[pl.ds(base, DCHUNK)], didx_v)
        pltpu.sync_copy(ones_v, deg_sh.at[didx_v], add=True)
        return 0

    lax.fori_loop(0, NDCHUNK, _chunk, 0)
    plsc.subcore_barrier()

    @pl.when(sid == 0)
    def _flush():
        pltpu.sync_copy(deg_sh, out_hbm.at[cid])


_deg_kernel = pl.kernel(
    _deg_body,
    out_type=jax.ShapeDtypeStruct((NC, N), jnp.float32),
    mesh=plsc.VectorSubcoreMesh(
        core_axis_name="c", subcore_axis_name="s", num_cores=NC,
        num_subcores=NS),
    scratch_types=[
        pltpu.VMEM((DCHUNK,), jnp.int32),
        pltpu.VMEM((DCHUNK,), jnp.float32),
        pltpu.VMEM_SHARED((N,), jnp.float32),
    ],
    compiler_params=pltpu.CompilerParams(use_tc_tiling_on_sc=False),
)


def _agg_body(g_hbm, src_hbm, dst_hbm, out_hbm, sidx_all, didx0, didx1,
              rows0, rows1, acc_sh, sem_g0, sem_g1, sem_d0, sem_d1):
    cid = lax.axis_index("c")
    sid = lax.axis_index("s")

    pltpu.sync_copy(src_hbm.at[pl.ds(sid * EPT, EPT)], sidx_all)

    def _zrow(i, _):
        rows0[i // 4, pl.ds((i % 4) * L, L)] = jnp.zeros((L,), jnp.float32)
        return 0

    lax.fori_loop(0, 16 * (HALF // L), _zrow, 0)

    def _zchunk(j, _):
        chunk = sid + j * NS

        @pl.when(chunk < N // 16)
        def _():
            off = pl.multiple_of(chunk * 16, 16)
            pltpu.sync_copy(rows0.at[pl.ds(0, 16)],
                            acc_sh.at[pl.ds(off, 16)])

        return 0

    lax.fori_loop(0, pl.cdiv(N // 16, NS), _zchunk, 0)

    ghalf = g_hbm.at[cid]

    def _gather(c, rows, sem):
        idx = sidx_all.at[pl.ds(c * CHUNK, CHUNK)]
        return pltpu.make_async_copy(ghalf.at[idx], rows, sem)

    def _didx(c, buf, sem):
        base = sid * EPT + c * CHUNK
        return pltpu.make_async_copy(dst_hbm.at[pl.ds(base, CHUNK)], buf,
                                     sem)

    _didx(0, didx0, sem_d0).start()
    _gather(0, rows0, sem_g0).start()
    plsc.subcore_barrier()

    def _pair(j, _):
        c0 = 2 * j
        c1 = 2 * j + 1
        _didx(c1, didx1, sem_d1).start()
        _gather(c1, rows1, sem_g1).start()
        _gather(c0, rows0, sem_g0).wait()
        _didx(c0, didx0, sem_d0).wait()
        pltpu.sync_copy(rows0, acc_sh.at[didx0], add=True)

        @pl.when(c1 + 1 < NCHUNK)
        def _():
            _didx(c1 + 1, didx0, sem_d0).start()
            _gather(c1 + 1, rows0, sem_g0).start()

        _gather(c1, rows1, sem_g1).wait()
        _didx(c1, didx1, sem_d1).wait()
        pltpu.sync_copy(rows1, acc_sh.at[didx1], add=True)
        return 0

    lax.fori_loop(0, NCHUNK // 2, _pair, 0)
    plsc.subcore_barrier()

    def _fchunk(j, _):
        chunk = sid + j * NS

        @pl.when(chunk < N // 16)
        def _():
            off = pl.multiple_of(chunk * 16, 16)
            pltpu.sync_copy(acc_sh.at[pl.ds(off, 16)],
                            out_hbm.at[cid, pl.ds(off, 16)])

        return 0

    lax.fori_loop(0, pl.cdiv(N // 16, NS), _fchunk, 0)


_agg_kernel = pl.kernel(
    _agg_body,
    out_type=jax.ShapeDtypeStruct((NC, N, HALF), jnp.float32),
    mesh=plsc.VectorSubcoreMesh(
        core_axis_name="c", subcore_axis_name="s", num_cores=NC,
        num_subcores=NS),
    scratch_types=[
        pltpu.VMEM((EPT,), jnp.int32),
        pltpu.VMEM((CHUNK,), jnp.int32),
        pltpu.VMEM((CHUNK,), jnp.int32),
        pltpu.VMEM((CHUNK, HALF), jnp.float32),
        pltpu.VMEM((CHUNK, HALF), jnp.float32),
        pltpu.VMEM_SHARED((N, HALF), jnp.float32),
        pltpu.SemaphoreType.DMA,
        pltpu.SemaphoreType.DMA,
        pltpu.SemaphoreType.DMA,
        pltpu.SemaphoreType.DMA,
    ],
    compiler_params=pltpu.CompilerParams(use_tc_tiling_on_sc=False),
)

MB = 2000
GRID = N // MB


def _k1_body(x_ref, w_ref, degp_ref, g_ref, dis_ref):
    deg = degp_ref[0] + degp_ref[1] + 1.0
    dis = lax.rsqrt(deg)
    h = jnp.dot(x_ref[...], w_ref[...],
                preferred_element_type=jnp.float32) * dis
    g_ref[0] = h[:, :HALF]
    g_ref[1] = h[:, HALF:]
    dis_ref[...] = dis


_k1 = pl.pallas_call(
    _k1_body,
    grid=(GRID,),
    in_specs=[
        pl.BlockSpec((MB, D), lambda i: (i, 0)),
        pl.BlockSpec((D, D), lambda i: (0, 0)),
        pl.BlockSpec((NC, MB, 1), lambda i: (0, i, 0)),
    ],
    out_specs=[
        pl.BlockSpec((NC, MB, HALF), lambda i: (0, i, 0)),
        pl.BlockSpec((MB, 1), lambda i: (i, 0)),
    ],
    out_shape=[
        jax.ShapeDtypeStruct((NC, N, HALF), jnp.float32),
        jax.ShapeDtypeStruct((N, 1), jnp.float32),
    ],
)


def _k2_body(s_ref, g_ref, dis_ref, b_ref, w_ref, g2_ref):
    sg = jnp.concatenate([s_ref[0] + g_ref[0], s_ref[1] + g_ref[1]],
                         axis=-1)
    t = jnp.maximum(sg * dis_ref[...] + b_ref[...], 0.0)
    h = jnp.dot(t, w_ref[...],
                preferred_element_type=jnp.float32) * dis_ref[...]
    g2_ref[0] = h[:, :HALF]
    g2_ref[1] = h[:, HALF:]


_k2 = pl.pallas_call(
    _k2_body,
    grid=(GRID,),
    in_specs=[
        pl.BlockSpec((NC, MB, HALF), lambda i: (0, i, 0)),
        pl.BlockSpec((NC, MB, HALF), lambda i: (0, i, 0)),
        pl.BlockSpec((MB, 1), lambda i: (i, 0)),
        pl.BlockSpec((1, D), lambda i: (0, 0)),
        pl.BlockSpec((D, D), lambda i: (0, 0)),
    ],
    out_specs=pl.BlockSpec((NC, MB, HALF), lambda i: (0, i, 0)),
    out_shape=jax.ShapeDtypeStruct((NC, N, HALF), jnp.float32),
)


def _k3_body(s_ref, g_ref, dis_ref, b_ref, w_ref, bo_ref, out_ref):
    sg = jnp.concatenate([s_ref[0] + g_ref[0], s_ref[1] + g_ref[1]],
                         axis=-1)
    t = sg * dis_ref[...] + b_ref[...]
    out_ref[...] = jnp.dot(
        t, w_ref[...], preferred_element_type=jnp.float32) + bo_ref[...]


_k3 = pl.pallas_call(
    _k3_body,
    grid=(GRID,),
    in_specs=[
        pl.BlockSpec((NC, MB, HALF), lambda i: (0, i, 0)),
        pl.BlockSpec((NC, MB, HALF), lambda i: (0, i, 0)),
        pl.BlockSpec((MB, 1), lambda i: (i, 0)),
        pl.BlockSpec((1, D), lambda i: (0, 0)),
        pl.BlockSpec((D, D), lambda i: (0, 0)),
        pl.BlockSpec((1, D), lambda i: (0, 0)),
    ],
    out_specs=pl.BlockSpec((MB, D), lambda i: (i, 0)),
    out_shape=jax.ShapeDtypeStruct((N, D), jnp.float32),
)


def kernel(x, edge_index, W1, b1, W2, b2, W_out, b_out):
    src = edge_index[0].astype(jnp.int32)
    dst = edge_index[1].astype(jnp.int32)

    degp = _deg_kernel(dst).reshape(NC, N, 1)
    g1, dis = _k1(x, W1, degp)
    s1 = _agg_kernel(g1, src, dst)
    g2 = _k2(s1, g1, dis, b1.reshape(1, D), W2)
    s2 = _agg_kernel(g2, src, dst)

    W_out_p = jnp.zeros((D, D), jnp.float32).at[:, :C_OUT].set(W_out)
    b_out_p = jnp.zeros((1, D), jnp.float32).at[0, :C_OUT].set(b_out)
    out = _k3(s2, g2, dis, b2.reshape(1, D), W_out_p, b_out_p)
    return out[:, :C_OUT]


# speedup vs baseline: 28.1344x; 1.0594x over previous
import jax
import jax.numpy as jnp
from jax import lax
from jax.experimental import pallas as pl
from jax.experimental.pallas import tpu as pltpu
from jax.experimental.pallas import tpu_sc as plsc

N = 10000
D = 128
C_OUT = 40
E = 320000

NC = 2
NS = 16
NW = NC * NS
L = 16

EPT = E // NW
CHUNK = 80
NCHUNK = EPT // CHUNK

DEPT = E // NW
DCHUNK = 2000
NDCHUNK = DEPT // DCHUNK


def _deg_body(dst_hbm, out_hbm, didx_v, ones_v, deg_sh):
    cid = lax.axis_index("c")
    sid = lax.axis_index("s")
    wid = cid * NS + sid

    def _fill_zero(i, _):
        ones_v[pl.ds(i * L, L)] = jnp.zeros((L,), jnp.float32)
        return 0

    def _fill_one(i, _):
        ones_v[pl.ds(i * L, L)] = jnp.ones((L,), jnp.float32)
        return 0

    lax.fori_loop(0, DCHUNK // L, _fill_zero, 0)

    @pl.when(sid == 0)
    def _zero_shared():
        for c in range(N // DCHUNK):
            pltpu.sync_copy(ones_v, deg_sh.at[pl.ds(c * DCHUNK, DCHUNK)])

    lax.fori_loop(0, DCHUNK // L, _fill_one, 0)
    plsc.subcore_barrier()

    def _chunk(c, _):
        base = wid * DEPT + c * DCHUNK
        pltpu.sync_copy(dst_hbm.at[pl.ds(base, DCHUNK)], didx_v)
        pltpu.sync_copy(ones_v, deg_sh.at[didx_v], add=True)
        return 0

    lax.fori_loop(0, NDCHUNK, _chunk, 0)
    plsc.subcore_barrier()

    @pl.when(sid == 0)
    def _flush():
        pltpu.sync_copy(deg_sh, out_hbm.at[cid])


_deg_kernel = pl.kernel(
    _deg_body,
    out_type=jax.ShapeDtypeStruct((NC, N), jnp.float32),
    mesh=plsc.VectorSubcoreMesh(
        core_axis_name="c", subcore_axis_name="s", num_cores=NC,
        num_subcores=NS),
    scratch_types=[
        pltpu.VMEM((DCHUNK,), jnp.int32),
        pltpu.VMEM((DCHUNK,), jnp.float32),
        pltpu.VMEM_SHARED((N,), jnp.float32),
    ],
)


def _agg_body(g_hbm, src_hbm, dst_hbm, out_hbm, sidx_all, didx0, didx1,
              rows0, rows1, acc_sh, sem_g0, sem_g1, sem_d0, sem_d1):
    cid = lax.axis_index("c")
    sid = lax.axis_index("s")
    wid = cid * NS + sid

    pltpu.sync_copy(src_hbm.at[pl.ds(wid * EPT, EPT)], sidx_all)

    def _zrow(i, _):
        rows0[i // 8, pl.ds((i % 8) * L, L)] = jnp.zeros((L,), jnp.float32)
        return 0

    lax.fori_loop(0, 16 * (D // L), _zrow, 0)

    def _zchunk(j, _):
        chunk = sid + j * NS

        @pl.when(chunk < N // 16)
        def _():
            off = pl.multiple_of(chunk * 16, 16)
            pltpu.sync_copy(rows0.at[pl.ds(0, 16)],
                            acc_sh.at[pl.ds(off, 16)])

        return 0

    lax.fori_loop(0, pl.cdiv(N // 16, NS), _zchunk, 0)

    def _gather(c, rows, sem):
        idx = sidx_all.at[pl.ds(c * CHUNK, CHUNK)]
        return pltpu.make_async_copy(g_hbm.at[idx], rows, sem)

    def _didx(c, buf, sem):
        base = wid * EPT + c * CHUNK
        return pltpu.make_async_copy(dst_hbm.at[pl.ds(base, CHUNK)], buf,
                                     sem)

    _didx(0, didx0, sem_d0).start()
    _gather(0, rows0, sem_g0).start()
    plsc.subcore_barrier()

    def _pair(j, _):
        c0 = 2 * j
        c1 = 2 * j + 1
        _didx(c1, didx1, sem_d1).start()
        _gather(c1, rows1, sem_g1).start()
        _gather(c0, rows0, sem_g0).wait()
        _didx(c0, didx0, sem_d0).wait()
        pltpu.sync_copy(rows0, acc_sh.at[didx0], add=True)

        @pl.when(c1 + 1 < NCHUNK)
        def _():
            _didx(c1 + 1, didx0, sem_d0).start()
            _gather(c1 + 1, rows0, sem_g0).start()

        _gather(c1, rows1, sem_g1).wait()
        _didx(c1, didx1, sem_d1).wait()
        pltpu.sync_copy(rows1, acc_sh.at[didx1], add=True)
        return 0

    lax.fori_loop(0, NCHUNK // 2, _pair, 0)

    _gather(NCHUNK - 1, rows0, sem_g0).wait()
    _didx(NCHUNK - 1, didx0, sem_d0).wait()
    pltpu.sync_copy(rows0, acc_sh.at[didx0], add=True)
    plsc.subcore_barrier()

    def _fchunk(j, _):
        chunk = sid + j * NS

        @pl.when(chunk < N // 16)
        def _():
            off = pl.multiple_of(chunk * 16, 16)
            pltpu.sync_copy(acc_sh.at[pl.ds(off, 16)],
                            out_hbm.at[cid, pl.ds(off, 16)])

        return 0

    lax.fori_loop(0, pl.cdiv(N // 16, NS), _fchunk, 0)


_agg_kernel = pl.kernel(
    _agg_body,
    out_type=jax.ShapeDtypeStruct((NC, N, D), jnp.float32),
    mesh=plsc.VectorSubcoreMesh(
        core_axis_name="c", subcore_axis_name="s", num_cores=NC,
        num_subcores=NS),
    scratch_types=[
        pltpu.VMEM((EPT,), jnp.int32),
        pltpu.VMEM((CHUNK,), jnp.int32),
        pltpu.VMEM((CHUNK,), jnp.int32),
        pltpu.VMEM((CHUNK, D), jnp.float32),
        pltpu.VMEM((CHUNK, D), jnp.float32),
        pltpu.VMEM_SHARED((N, D), jnp.float32),
        pltpu.SemaphoreType.DMA,
        pltpu.SemaphoreType.DMA,
        pltpu.SemaphoreType.DMA,
        pltpu.SemaphoreType.DMA,
    ],
)

MB = 2000
GRID = N // MB


def _k1_body(x_ref, w_ref, degp_ref, g_ref, dis_ref):
    deg = degp_ref[0] + degp_ref[1] + 1.0
    dis = lax.rsqrt(deg)
    h = jnp.dot(x_ref[...], w_ref[...],
                preferred_element_type=jnp.float32)
    g_ref[...] = h * dis
    dis_ref[...] = dis


_k1 = pl.pallas_call(
    _k1_body,
    grid=(GRID,),
    in_specs=[
        pl.BlockSpec((MB, D), lambda i: (i, 0)),
        pl.BlockSpec((D, D), lambda i: (0, 0)),
        pl.BlockSpec((NC, MB, 1), lambda i: (0, i, 0)),
    ],
    out_specs=[
        pl.BlockSpec((MB, D), lambda i: (i, 0)),
        pl.BlockSpec((MB, 1), lambda i: (i, 0)),
    ],
    out_shape=[
        jax.ShapeDtypeStruct((N, D), jnp.float32),
        jax.ShapeDtypeStruct((N, 1), jnp.float32),
    ],
)


def _k2_body(s_ref, g_ref, dis_ref, b_ref, w_ref, g2_ref):
    t = (s_ref[0] + s_ref[1] + g_ref[...]) * dis_ref[...] + b_ref[...]
    t = jnp.maximum(t, 0.0)
    g2_ref[...] = jnp.dot(
        t, w_ref[...], preferred_element_type=jnp.float32) * dis_ref[...]


_k2 = pl.pallas_call(
    _k2_body,
    grid=(GRID,),
    in_specs=[
        pl.BlockSpec((NC, MB, D), lambda i: (0, i, 0)),
        pl.BlockSpec((MB, D), lambda i: (i, 0)),
        pl.BlockSpec((MB, 1), lambda i: (i, 0)),
        pl.BlockSpec((1, D), lambda i: (0, 0)),
        pl.BlockSpec((D, D), lambda i: (0, 0)),
    ],
    out_specs=pl.BlockSpec((MB, D), lambda i: (i, 0)),
    out_shape=jax.ShapeDtypeStruct((N, D), jnp.float32),
)


def _k3_body(s_ref, g_ref, dis_ref, b_ref, w_ref, bo_ref, out_ref):
    t = (s_ref[0] + s_ref[1] + g_ref[...]) * dis_ref[...] + b_ref[...]
    out_ref[...] = jnp.dot(
        t, w_ref[...], preferred_element_type=jnp.float32) + bo_ref[...]


_k3 = pl.pallas_call(
    _k3_body,
    grid=(GRID,),
    in_specs=[
        pl.BlockSpec((NC, MB, D), lambda i: (0, i, 0)),
        pl.BlockSpec((MB, D), lambda i: (i, 0)),
        pl.BlockSpec((MB, 1), lambda i: (i, 0)),
        pl.BlockSpec((1, D), lambda i: (0, 0)),
        pl.BlockSpec((D, D), lambda i: (0, 0)),
        pl.BlockSpec((1, D), lambda i: (0, 0)),
    ],
    out_specs=pl.BlockSpec((MB, D), lambda i: (i, 0)),
    out_shape=jax.ShapeDtypeStruct((N, D), jnp.float32),
)


def kernel(x, edge_index, W1, b1, W2, b2, W_out, b_out):
    src = edge_index[0].astype(jnp.int32)
    dst = edge_index[1].astype(jnp.int32)

    degp = _deg_kernel(dst).reshape(NC, N, 1)
    g1, dis = _k1(x, W1, degp)
    s1 = _agg_kernel(g1, src, dst)
    g2 = _k2(s1, g1, dis, b1.reshape(1, D), W2)
    s2 = _agg_kernel(g2, src, dst)

    W_out_p = jnp.zeros((D, D), jnp.float32).at[:, :C_OUT].set(W_out)
    b_out_p = jnp.zeros((1, D), jnp.float32).at[0, :C_OUT].set(b_out)
    out = _k3(s2, g2, dis, b2.reshape(1, D), W_out_p, b_out_p)
    return out[:, :C_OUT]


# speedup vs baseline: 33.5425x; 1.1922x over previous
import jax
import jax.numpy as jnp
from jax import lax
from jax.experimental import pallas as pl
from jax.experimental.pallas import tpu as pltpu
from jax.experimental.pallas import tpu_sc as plsc

N = 10000
D = 128
C_OUT = 40
E = 320000

NC = 2
NS = 16
NW = NC * NS
L = 16

EPT = E // NW
CHUNK = 80
NCHUNK = EPT // CHUNK
NBUF = 4

DEPT = E // NW
DCHUNK = 2000
NDCHUNK = DEPT // DCHUNK


def _deg_body(edge_hbm, out_hbm, didx_v, ones_v, deg_sh):
    cid = lax.axis_index("c")
    sid = lax.axis_index("s")
    wid = cid * NS + sid

    def _fill_zero(i, _):
        ones_v[pl.ds(i * L, L)] = jnp.zeros((L,), jnp.float32)
        return 0

    def _fill_one(i, _):
        ones_v[pl.ds(i * L, L)] = jnp.ones((L,), jnp.float32)
        return 0

    lax.fori_loop(0, DCHUNK // L, _fill_zero, 0)

    @pl.when(sid == 0)
    def _zero_shared():
        for c in range(N // DCHUNK):
            pltpu.sync_copy(ones_v, deg_sh.at[pl.ds(c * DCHUNK, DCHUNK)])

    lax.fori_loop(0, DCHUNK // L, _fill_one, 0)
    plsc.subcore_barrier()

    def _chunk(c, _):
        base = E + wid * DEPT + c * DCHUNK
        pltpu.sync_copy(edge_hbm.at[pl.ds(base, DCHUNK)], didx_v)
        pltpu.sync_copy(ones_v, deg_sh.at[didx_v], add=True)
        return 0

    lax.fori_loop(0, NDCHUNK, _chunk, 0)
    plsc.subcore_barrier()

    @pl.when(sid == 0)
    def _flush():
        pltpu.sync_copy(deg_sh, out_hbm.at[cid])


_deg_kernel = pl.kernel(
    _deg_body,
    out_type=jax.ShapeDtypeStruct((NC, N), jnp.float32),
    mesh=plsc.VectorSubcoreMesh(
        core_axis_name="c", subcore_axis_name="s", num_cores=NC,
        num_subcores=NS),
    scratch_types=[
        pltpu.VMEM((DCHUNK,), jnp.int32),
        pltpu.VMEM((DCHUNK,), jnp.float32),
        pltpu.VMEM_SHARED((N,), jnp.float32),
    ],
)


def _agg_body(g_hbm, edge_hbm, out_hbm, sb0, sb1, sb2, sb3, db0, db1, db2,
              db3, rw0, rw1, rw2, rw3, acc_sh, se0, se1, se2, se3, sg0,
              sg1, sg2, sg3):
    cid = lax.axis_index("c")
    sid = lax.axis_index("s")
    wid = cid * NS + sid

    sbufs = (sb0, sb1, sb2, sb3)
    dbufs = (db0, db1, db2, db3)
    rows = (rw0, rw1, rw2, rw3)
    sems_e = (se0, se1, se2, se3)
    sems_g = (sg0, sg1, sg2, sg3)

    def _sidx(c, b):
        base = wid * EPT + c * CHUNK
        return pltpu.make_async_copy(edge_hbm.at[pl.ds(base, CHUNK)],
                                     sbufs[b], sems_e[b])

    def _didx(c, b):
        base = E + wid * EPT + c * CHUNK
        return pltpu.make_async_copy(edge_hbm.at[pl.ds(base, CHUNK)],
                                     dbufs[b], sems_e[b])

    def _gather(b):
        return pltpu.make_async_copy(g_hbm.at[sbufs[b]], rows[b],
                                     sems_g[b])

    def _zrow(i, _):
        rw0[i // 8, pl.ds((i % 8) * L, L)] = jnp.zeros((L,), jnp.float32)
        return 0

    lax.fori_loop(0, 16 * (D // L), _zrow, 0)

    def _zchunk(j, _):
        chunk = sid + j * NS

        @pl.when(chunk < N // 16)
        def _():
            off = pl.multiple_of(chunk * 16, 16)
            pltpu.sync_copy(rw0.at[pl.ds(0, 16)],
                            acc_sh.at[pl.ds(off, 16)])

        return 0

    lax.fori_loop(0, pl.cdiv(N // 16, NS), _zchunk, 0)

    for b in range(NBUF):
        _sidx(b, b).start()
        _didx(b, b).start()
    for c in range(2):
        _sidx(c, c).wait()
        _didx(c, c).wait()
        _gather(c).start()
    plsc.subcore_barrier()

    def _step(c, b):
        @pl.when(c < NCHUNK)
        def _():
            b2 = (b + 2) % NBUF

            @pl.when(c + 2 < NCHUNK)
            def _():
                _sidx(c + 2, b2).wait()
                _didx(c + 2, b2).wait()
                _gather(b2).start()

            _gather(b).wait()
            pltpu.sync_copy(rows[b], acc_sh.at[dbufs[b]], add=True)

            @pl.when(c + NBUF < NCHUNK)
            def _():
                _sidx(c + NBUF, b).start()
                _didx(c + NBUF, b).start()

    def _quad(q, _):
        for u in range(NBUF):
            _step(q * NBUF + u, u)
        return 0

    lax.fori_loop(0, pl.cdiv(NCHUNK, NBUF), _quad, 0)
    plsc.subcore_barrier()

    def _fchunk(j, _):
        chunk = sid + j * NS

        @pl.when(chunk < N // 16)
        def _():
            off = pl.multiple_of(chunk * 16, 16)
            pltpu.sync_copy(acc_sh.at[pl.ds(off, 16)],
                            out_hbm.at[cid, pl.ds(off, 16)])

        return 0

    lax.fori_loop(0, pl.cdiv(N // 16, NS), _fchunk, 0)


_agg_kernel = pl.kernel(
    _agg_body,
    out_type=jax.ShapeDtypeStruct((NC, N, D), jnp.float32),
    mesh=plsc.VectorSubcoreMesh(
        core_axis_name="c", subcore_axis_name="s", num_cores=NC,
        num_subcores=NS),
    scratch_types=(
        [pltpu.VMEM((CHUNK,), jnp.int32) for _ in range(2 * NBUF)]
        + [pltpu.VMEM((CHUNK, D), jnp.float32) for _ in range(NBUF)]
        + [pltpu.VMEM_SHARED((N, D), jnp.float32)]
        + [pltpu.SemaphoreType.DMA for _ in range(2 * NBUF)]
    ),
)

MB = 2000
GRID = N // MB


def _k1_body(x_ref, w_ref, degp_ref, g_ref, dis_ref):
    deg = degp_ref[0] + degp_ref[1] + 1.0
    dis = lax.rsqrt(deg)
    h = jnp.dot(x_ref[...], w_ref[...],
                preferred_element_type=jnp.float32)
    g_ref[...] = h * dis
    dis_ref[...] = dis


_k1 = pl.pallas_call(
    _k1_body,
    grid=(GRID,),
    in_specs=[
        pl.BlockSpec((MB, D), lambda i: (i, 0)),
        pl.BlockSpec((D, D), lambda i: (0, 0)),
        pl.BlockSpec((NC, MB, 1), lambda i: (0, i, 0)),
    ],
    out_specs=[
        pl.BlockSpec((MB, D), lambda i: (i, 0)),
        pl.BlockSpec((MB, 1), lambda i: (i, 0)),
    ],
    out_shape=[
        jax.ShapeDtypeStruct((N, D), jnp.float32),
        jax.ShapeDtypeStruct((N, 1), jnp.float32),
    ],
)


def _k2_body(s_ref, g_ref, dis_ref, b_ref, w_ref, g2_ref):
    t = (s_ref[0] + s_ref[1] + g_ref[...]) * dis_ref[...] + b_ref[...]
    t = jnp.maximum(t, 0.0)
    g2_ref[...] = jnp.dot(
        t, w_ref[...], preferred_element_type=jnp.float32) * dis_ref[...]


_k2 = pl.pallas_call(
    _k2_body,
    grid=(GRID,),
    in_specs=[
        pl.BlockSpec((NC, MB, D), lambda i: (0, i, 0)),
        pl.BlockSpec((MB, D), lambda i: (i, 0)),
        pl.BlockSpec((MB, 1), lambda i: (i, 0)),
        pl.BlockSpec((1, D), lambda i: (0, 0)),
        pl.BlockSpec((D, D), lambda i: (0, 0)),
    ],
    out_specs=pl.BlockSpec((MB, D), lambda i: (i, 0)),
    out_shape=jax.ShapeDtypeStruct((N, D), jnp.float32),
)


def _k3_body(s_ref, g_ref, dis_ref, b_ref, w_ref, bo_ref, out_ref):
    t = (s_ref[0] + s_ref[1] + g_ref[...]) * dis_ref[...] + b_ref[...]
    out_ref[...] = jnp.dot(
        t, w_ref[...], preferred_element_type=jnp.float32) + bo_ref[...]


_k3 = pl.pallas_call(
    _k3_body,
    grid=(GRID,),
    in_specs=[
        pl.BlockSpec((NC, MB, D), lambda i: (0, i, 0)),
        pl.BlockSpec((MB, D), lambda i: (i, 0)),
        pl.BlockSpec((MB, 1), lambda i: (i, 0)),
        pl.BlockSpec((1, D), lambda i: (0, 0)),
        pl.BlockSpec((D, C_OUT), lambda i: (0, 0)),
        pl.BlockSpec((1, C_OUT), lambda i: (0, 0)),
    ],
    out_specs=pl.BlockSpec((MB, C_OUT), lambda i: (i, 0)),
    out_shape=jax.ShapeDtypeStruct((N, C_OUT), jnp.float32),
)


def kernel(x, edge_index, W1, b1, W2, b2, W_out, b_out):
    edge1d = edge_index.astype(jnp.int32).reshape(2 * E)

    degp = _deg_kernel(edge1d).reshape(NC, N, 1)
    g1, dis = _k1(x, W1, degp)
    s1 = _agg_kernel(g1, edge1d)
    g2 = _k2(s1, g1, dis, b1.reshape(1, D), W2)
    s2 = _agg_kernel(g2, edge1d)
    out = _k3(s2, g2, dis, b2.reshape(1, D), W_out,
              b_out.reshape(1, C_OUT))
    return out


# speedup vs baseline: 38.2746x; 1.1411x over previous
import jax
import jax.numpy as jnp
from jax import lax
from jax.experimental import pallas as pl
from jax.experimental.pallas import tpu as pltpu
from jax.experimental.pallas import tpu_sc as plsc

N = 10000
D = 128
C_OUT = 40
E = 320000

NC = 2
NS = 16
NW = NC * NS
L = 16

EPT = E // NW
CHUNK = 80
NCHUNK = EPT // CHUNK
NBUF = 4

DEPT = E // NW
DCHUNK = 2000
NDCHUNK = DEPT // DCHUNK


def _deg_body(edge_hbm, out_hbm, didx_v, ones_v, deg_sh):
    cid = lax.axis_index("c")
    sid = lax.axis_index("s")
    wid = cid * NS + sid

    def _fill_zero(i, _):
        ones_v[pl.ds(i * L, L)] = jnp.zeros((L,), jnp.float32)
        return 0

    def _fill_one(i, _):
        ones_v[pl.ds(i * L, L)] = jnp.ones((L,), jnp.float32)
        return 0

    lax.fori_loop(0, DCHUNK // L, _fill_zero, 0)

    @pl.when(sid == 0)
    def _zero_shared():
        for c in range(N // DCHUNK):
            pltpu.sync_copy(ones_v, deg_sh.at[pl.ds(c * DCHUNK, DCHUNK)])

    lax.fori_loop(0, DCHUNK // L, _fill_one, 0)
    plsc.subcore_barrier()

    def _chunk(c, _):
        base = E + wid * DEPT + c * DCHUNK
        pltpu.sync_copy(edge_hbm.at[pl.ds(base, DCHUNK)], didx_v)
        pltpu.sync_copy(ones_v, deg_sh.at[didx_v], add=True)
        return 0

    lax.fori_loop(0, NDCHUNK, _chunk, 0)
    plsc.subcore_barrier()

    @pl.when(sid == 0)
    def _flush():
        pltpu.sync_copy(deg_sh, out_hbm.at[cid])


_deg_kernel = pl.kernel(
    _deg_body,
    out_type=jax.ShapeDtypeStruct((NC, N), jnp.float32),
    mesh=plsc.VectorSubcoreMesh(
        core_axis_name="c", subcore_axis_name="s", num_cores=NC,
        num_subcores=NS),
    scratch_types=[
        pltpu.VMEM((DCHUNK,), jnp.int32),
        pltpu.VMEM((DCHUNK,), jnp.float32),
        pltpu.VMEM_SHARED((N,), jnp.float32),
    ],
)


def _agg_body(g_hbm, edge_hbm, out_hbm, sb0, sb1, sb2, sb3, db0, db1, db2,
              db3, rw0, rw1, rw2, rw3, acc_sh, se0, se1, se2, se3, sg0,
              sg1, sg2, sg3):
    cid = lax.axis_index("c")
    sid = lax.axis_index("s")
    wid = cid * NS + sid

    sbufs = (sb0, sb1, sb2, sb3)
    dbufs = (db0, db1, db2, db3)
    rows = (rw0, rw1, rw2, rw3)
    sems_e = (se0, se1, se2, se3)
    sems_g = (sg0, sg1, sg2, sg3)

    def _sidx(c, b):
        base = wid * EPT + c * CHUNK
        return pltpu.make_async_copy(edge_hbm.at[pl.ds(base, CHUNK)],
                                     sbufs[b], sems_e[b])

    def _didx(c, b):
        base = E + wid * EPT + c * CHUNK
        return pltpu.make_async_copy(edge_hbm.at[pl.ds(base, CHUNK)],
                                     dbufs[b], sems_e[b])

    def _gather(b):
        return pltpu.make_async_copy(g_hbm.at[sbufs[b]], rows[b],
                                     sems_g[b])

    def _zrow(i, _):
        rw0[i // 8, pl.ds((i % 8) * L, L)] = jnp.zeros((L,), jnp.float32)
        return 0

    lax.fori_loop(0, 16 * (D // L), _zrow, 0)

    def _zchunk(j, _):
        chunk = sid + j * NS

        @pl.when(chunk < N // 16)
        def _():
            off = pl.multiple_of(chunk * 16, 16)
            pltpu.sync_copy(rw0.at[pl.ds(0, 16)],
                            acc_sh.at[pl.ds(off, 16)])

        return 0

    lax.fori_loop(0, pl.cdiv(N // 16, NS), _zchunk, 0)

    for b in range(NBUF):
        _sidx(b, b).start()
        _didx(b, b).start()
    for c in range(2):
        _sidx(c, c).wait()
        _didx(c, c).wait()
        _gather(c).start()
    plsc.subcore_barrier()

    def _step(c, b):
        @pl.when(c < NCHUNK)
        def _():
            b2 = (b + 2) % NBUF

            @pl.when(c + 2 < NCHUNK)
            def _():
                _sidx(c + 2, b2).wait()
                _didx(c + 2, b2).wait()
                _gather(b2).start()

            _gather(b).wait()
            pltpu.sync_copy(rows[b], acc_sh.at[dbufs[b]], add=True)

            @pl.when(c + NBUF < NCHUNK)
            def _():
                _sidx(c + NBUF, b).start()
                _didx(c + NBUF, b).start()

    def _quad(q, _):
        for u in range(NBUF):
            _step(q * NBUF + u, u)
        return 0

    lax.fori_loop(0, pl.cdiv(NCHUNK, NBUF), _quad, 0)
    plsc.subcore_barrier()

    FR = 624
    foff = pl.multiple_of(sid * FR, 8)
    pltpu.sync_copy(acc_sh.at[pl.ds(foff, FR)],
                    out_hbm.at[cid, pl.ds(foff, FR)])

    @pl.when(sid == 0)
    def _frem():
        pltpu.sync_copy(acc_sh.at[pl.ds(NS * FR, N - NS * FR)],
                        out_hbm.at[cid, pl.ds(NS * FR, N - NS * FR)])


_agg_kernel = pl.kernel(
    _agg_body,
    out_type=jax.ShapeDtypeStruct((NC, N, D), jnp.float32),
    mesh=plsc.VectorSubcoreMesh(
        core_axis_name="c", subcore_axis_name="s", num_cores=NC,
        num_subcores=NS),
    scratch_types=(
        [pltpu.VMEM((CHUNK,), jnp.int32) for _ in range(2 * NBUF)]
        + [pltpu.VMEM((CHUNK, D), jnp.float32) for _ in range(NBUF)]
        + [pltpu.VMEM_SHARED((N, D), jnp.float32)]
        + [pltpu.SemaphoreType.DMA for _ in range(2 * NBUF)]
    ),
)

MB = 2000
GRID = N // MB


def _k1_body(x_ref, w_ref, degp_ref, g_ref, dis_ref):
    deg = degp_ref[0] + degp_ref[1] + 1.0
    dis = lax.rsqrt(deg)
    h = jnp.dot(x_ref[...], w_ref[...],
                preferred_element_type=jnp.float32)
    g_ref[...] = h * dis
    dis_ref[...] = dis


_k1 = pl.pallas_call(
    _k1_body,
    grid=(GRID,),
    in_specs=[
        pl.BlockSpec((MB, D), lambda i: (i, 0)),
        pl.BlockSpec((D, D), lambda i: (0, 0)),
        pl.BlockSpec((NC, MB, 1), lambda i: (0, i, 0)),
    ],
    out_specs=[
        pl.BlockSpec((MB, D), lambda i: (i, 0)),
        pl.BlockSpec((MB, 1), lambda i: (i, 0)),
    ],
    out_shape=[
        jax.ShapeDtypeStruct((N, D), jnp.float32),
        jax.ShapeDtypeStruct((N, 1), jnp.float32),
    ],
)


def _k2_body(s_ref, g_ref, dis_ref, b_ref, w_ref, g2_ref):
    t = (s_ref[0] + s_ref[1] + g_ref[...]) * dis_ref[...] + b_ref[...]
    t = jnp.maximum(t, 0.0)
    g2_ref[...] = jnp.dot(
        t, w_ref[...], preferred_element_type=jnp.float32) * dis_ref[...]


_k2 = pl.pallas_call(
    _k2_body,
    grid=(GRID,),
    in_specs=[
        pl.BlockSpec((NC, MB, D), lambda i: (0, i, 0)),
        pl.BlockSpec((MB, D), lambda i: (i, 0)),
        pl.BlockSpec((MB, 1), lambda i: (i, 0)),
        pl.BlockSpec((1, D), lambda i: (0, 0)),
        pl.BlockSpec((D, D), lambda i: (0, 0)),
    ],
    out_specs=pl.BlockSpec((MB, D), lambda i: (i, 0)),
    out_shape=jax.ShapeDtypeStruct((N, D), jnp.float32),
)


def _k3_body(s_ref, g_ref, dis_ref, b_ref, w_ref, bo_ref, out_ref):
    t = (s_ref[0] + s_ref[1] + g_ref[...]) * dis_ref[...] + b_ref[...]
    out_ref[...] = jnp.dot(
        t, w_ref[...], preferred_element_type=jnp.float32) + bo_ref[...]


_k3 = pl.pallas_call(
    _k3_body,
    grid=(GRID,),
    in_specs=[
        pl.BlockSpec((NC, MB, D), lambda i: (0, i, 0)),
        pl.BlockSpec((MB, D), lambda i: (i, 0)),
        pl.BlockSpec((MB, 1), lambda i: (i, 0)),
        pl.BlockSpec((1, D), lambda i: (0, 0)),
        pl.BlockSpec((D, C_OUT), lambda i: (0, 0)),
        pl.BlockSpec((1, C_OUT), lambda i: (0, 0)),
    ],
    out_specs=pl.BlockSpec((MB, C_OUT), lambda i: (i, 0)),
    out_shape=jax.ShapeDtypeStruct((N, C_OUT), jnp.float32),
)


def kernel(x, edge_index, W1, b1, W2, b2, W_out, b_out):
    edge1d = edge_index.astype(jnp.int32).reshape(2 * E)

    degp = _deg_kernel(edge1d).reshape(NC, N, 1)
    g1, dis = _k1(x, W1, degp)
    s1 = _agg_kernel(g1, edge1d)
    g2 = _k2(s1, g1, dis, b1.reshape(1, D), W2)
    s2 = _agg_kernel(g2, edge1d)
    out = _k3(s2, g2, dis, b2.reshape(1, D), W_out,
              b_out.reshape(1, C_OUT))
    return out
